# baseline (speedup 1.0000x reference)
import jax
import jax.numpy as jnp
from jax import lax
from jax.experimental import pallas as pl
from jax.experimental.pallas import tpu as pltpu
from jax.experimental.pallas import tpu_sc as plsc

D_MODEL = 768
N_PATCHES = 576
N_TOT = 577
BATCH = 128
NW = 32
RPW = N_PATCHES // NW
CHUNK = RPW * D_MODEL
LANES = 16
BPW = BATCH // NW

_INTERPRET = False


def _sc_body(in_hbm, cls_hbm, pos_hbm, out_hbm,
             pos_v, row0_v, cls_v, in_v, out_v):
    c = lax.axis_index("c")
    s = lax.axis_index("s")
    wid = s * 2 + c

    pos_off = pl.multiple_of((1 + wid * RPW) * D_MODEL, 256)
    pltpu.sync_copy(pos_hbm.at[pl.ds(pos_off, CHUNK)], pos_v)

    pltpu.sync_copy(cls_hbm.at[pl.ds(0, D_MODEL)], cls_v)
    pltpu.sync_copy(pos_hbm.at[pl.ds(0, D_MODEL)], row0_v)

    @pl.loop(0, D_MODEL // LANES)
    def _row0(i):
        o = i * LANES
        row0_v[pl.ds(o, LANES)] = row0_v[pl.ds(o, LANES)] + cls_v[pl.ds(o, LANES)]

    @pl.loop(0, BPW)
    def _cls_rows(j):
        b = wid * BPW + j
        off = pl.multiple_of(b * N_TOT * D_MODEL, 256)
        pltpu.sync_copy(row0_v, out_hbm.at[pl.ds(off, D_MODEL)])

    @pl.loop(0, BATCH)
    def _batch(b):
        in_off = pl.multiple_of(b * N_PATCHES * D_MODEL + wid * CHUNK, 256)
        pltpu.sync_copy(in_hbm.at[pl.ds(in_off, CHUNK)], in_v)

        @pl.loop(0, CHUNK // LANES, unroll=8)
        def _add(i):
            o = i * LANES
            out_v[pl.ds(o, LANES)] = in_v[pl.ds(o, LANES)] + pos_v[pl.ds(o, LANES)]

        out_off = pl.multiple_of(b * N_TOT * D_MODEL + (1 + wid * RPW) * D_MODEL, 256)
        pltpu.sync_copy(out_v, out_hbm.at[pl.ds(out_off, CHUNK)])


def kernel(inputs, class_embed, pos_table):
    in_flat = inputs.reshape(-1)
    cls_flat = class_embed.reshape(-1)
    pos_flat = pos_table.reshape(-1)
    mesh = plsc.VectorSubcoreMesh(
        core_axis_name="c", subcore_axis_name="s", num_cores=2, num_subcores=16)
    out_flat = pl.kernel(
        _sc_body,
        interpret=_INTERPRET,
        out_type=jax.ShapeDtypeStruct((BATCH * N_TOT * D_MODEL,), jnp.float32),
        mesh=mesh,
        scratch_types=[
            pltpu.VMEM((CHUNK,), jnp.float32),
            pltpu.VMEM((D_MODEL,), jnp.float32),
            pltpu.VMEM((D_MODEL,), jnp.float32),
            pltpu.VMEM((CHUNK,), jnp.float32),
            pltpu.VMEM((CHUNK,), jnp.float32),
        ],
    )(in_flat, cls_flat, pos_flat)
    return out_flat.reshape(BATCH, N_TOT, D_MODEL)


# speedup vs baseline: 5.9568x; 5.9568x over previous
import jax
import jax.numpy as jnp
from jax import lax
from jax.experimental import pallas as pl
from jax.experimental.pallas import tpu as pltpu
from jax.experimental.pallas import tpu_sc as plsc

D_MODEL = 768
N_PATCHES = 576
N_TOT = 577
BATCH = 128
NW = 32
LANES = 16
PB = N_PATCHES // 8
TB = BATCH // 8
TC = D_MODEL // 128
UNITS = PB * TB * TC
UPW = UNITS // NW


def _sc_body(in_hbm, pos_hbm, row0_hbm, out_hbm,
             row0_v, row0_rep, in_bufs, pos_bufs, out_bufs,
             in_sems, pos_sems, out_sems):
    c = lax.axis_index("c")
    s = lax.axis_index("s")
    wid = s * 2 + c

    def unit(j):
        u = wid + NW * j
        pb = u // (TB * TC)
        rem = u - pb * (TB * TC)
        tb = rem // TC
        tc = rem - tb * TC
        return pb, tb, tc

    def in_src(j):
        pb, tb, tc = unit(j)
        return in_hbm.at[pl.ds(8 * tb, 8), pl.ds(8 * pb, 8), pl.ds(128 * tc, 128)]

    def pos_src(j):
        pb, _, tc = unit(j)
        return pos_hbm.at[pl.ds(8 * pb, 8), pl.ds(128 * tc, 128)]

    def out_dst(j):
        pb, tb, tc = unit(j)
        return out_hbm.at[pl.ds(8 * pb + 1, 8), pl.ds(8 * tb, 8), pl.ds(128 * tc, 128)]

    pltpu.sync_copy(row0_hbm, row0_v)
    for r in range(8):
        @plsc.parallel_loop(0, D_MODEL // LANES, unroll=4)
        def _rep(i):
            o = i * LANES
            row0_rep[r, pl.ds(o, LANES)] = row0_v[pl.ds(o, LANES)]

    @pl.when(wid < TB)
    def _cls_row():
        pltpu.sync_copy(row0_rep, out_hbm.at[0, pl.ds(8 * wid, 8), :])

    for ph in range(2):
        pltpu.async_copy(in_src(ph), in_bufs.at[ph], in_sems.at[ph])
        pltpu.async_copy(pos_src(ph), pos_bufs.at[ph], pos_sems.at[ph])

    @pl.loop(0, UPW, step=2)
    def _unit_loop(g):
        for ph in range(2):
            j = g + ph
            in_buf = in_bufs.at[ph]
            pos_buf = pos_bufs.at[ph]
            out_buf = out_bufs.at[ph]
            pltpu.make_async_copy(in_src(j), in_buf, in_sems.at[ph]).wait()
            pltpu.make_async_copy(pos_src(j), pos_buf, pos_sems.at[ph]).wait()

            @pl.when(g > 0)
            def _wait_out():
                pltpu.make_async_copy(out_buf, out_dst(j), out_sems.at[ph]).wait()

            for bb in range(8):
                @plsc.parallel_loop(0, 64, unroll=8)
                def _add(i):
                    pp = i >> 3
                    o = (i & 7) * LANES
                    out_buf[pp, bb, pl.ds(o, LANES)] = (
                        in_buf[bb, pp, pl.ds(o, LANES)]
                        + pos_buf[pp, pl.ds(o, LANES)])

            pltpu.async_copy(out_buf, out_dst(j), out_sems.at[ph])
            nj = jnp.minimum(j + 2, UPW - 1)
            pltpu.async_copy(in_src(nj), in_bufs.at[ph], in_sems.at[ph])
            pltpu.async_copy(pos_src(nj), pos_bufs.at[ph], pos_sems.at[ph])

    for ph in range(2):
        pltpu.make_async_copy(in_src(UPW - 1), in_bufs.at[ph], in_sems.at[ph]).wait()
        pltpu.make_async_copy(pos_src(UPW - 1), pos_bufs.at[ph], pos_sems.at[ph]).wait()
        pltpu.make_async_copy(out_bufs.at[ph], out_dst(UPW - 2 + ph), out_sems.at[ph]).wait()


def kernel(inputs, class_embed, pos_table):
    pos_sh = pos_table[1:]
    row0 = class_embed.reshape(D_MODEL) + pos_table[0]
    mesh = plsc.VectorSubcoreMesh(
        core_axis_name="c", subcore_axis_name="s", num_cores=2, num_subcores=16)
    out_phys = pl.kernel(
        _sc_body,
        out_type=jax.ShapeDtypeStruct((N_TOT, BATCH, D_MODEL), jnp.float32),
        mesh=mesh,
        scratch_types=[
            pltpu.VMEM((D_MODEL,), jnp.float32),
            pltpu.VMEM((8, D_MODEL), jnp.float32),
            pltpu.VMEM((2, 8, 8, 128), jnp.float32),
            pltpu.VMEM((2, 8, 128), jnp.float32),
            pltpu.VMEM((2, 8, 8, 128), jnp.float32),
            pltpu.SemaphoreType.DMA((2,)),
            pltpu.SemaphoreType.DMA((2,)),
            pltpu.SemaphoreType.DMA((2,)),
        ],
    )(inputs, pos_sh, row0)
    return jnp.transpose(out_phys, (1, 0, 2))


# speedup vs baseline: 7.5537x; 1.2681x over previous
import jax
import jax.numpy as jnp
from jax import lax
from jax.experimental import pallas as pl
from jax.experimental.pallas import tpu as pltpu
from jax.experimental.pallas import tpu_sc as plsc

D_MODEL = 768
N_PATCHES = 576
N_TOT = 577
BATCH = 128
NW = 32
LANES = 16
W = 256
WV = W // LANES
PB = N_PATCHES // 8
TB = BATCH // 8
TC = D_MODEL // W
UNITS = PB * TB * TC
UPW = UNITS // NW
NBUF = 3


def _sc_body(in_hbm, pos_hbm, row0_hbm, out_hbm,
             row0_v, row0_rep, in_bufs, pos_bufs, out_bufs,
             in_sems, pos_sems, out_sems):
    c = lax.axis_index("c")
    s = lax.axis_index("s")
    wid = s * 2 + c

    def unit(j):
        u = wid + NW * j
        pb = u // (TB * TC)
        rem = u - pb * (TB * TC)
        tb = rem // TC
        tc = rem - tb * TC
        return pb, tb, tc

    def in_src(j):
        pb, tb, tc = unit(j)
        return in_hbm.at[pl.ds(8 * tb, 8), pl.ds(8 * pb, 8), pl.ds(W * tc, W)]

    def pos_src(j):
        pb, _, tc = unit(j)
        return pos_hbm.at[pl.ds(8 * pb, 8), pl.ds(W * tc, W)]

    def out_dst(j):
        pb, tb, tc = unit(j)
        return out_hbm.at[pl.ds(8 * pb + 1, 8), pl.ds(8 * tb, 8), pl.ds(W * tc, W)]

    pltpu.sync_copy(row0_hbm, row0_v)
    for r in range(8):
        @plsc.parallel_loop(0, D_MODEL // LANES, unroll=4)
        def _rep(i):
            o = i * LANES
            row0_rep[r, pl.ds(o, LANES)] = row0_v[pl.ds(o, LANES)]

    @pl.when(wid < TB)
    def _cls_row():
        pltpu.sync_copy(row0_rep, out_hbm.at[0, pl.ds(8 * wid, 8), :])

    for ph in range(NBUF):
        pltpu.async_copy(in_src(ph), in_bufs.at[ph], in_sems.at[ph])
        pltpu.async_copy(pos_src(ph), pos_bufs.at[ph], pos_sems.at[ph])

    @pl.loop(0, UPW, step=NBUF)
    def _unit_loop(g):
        for ph in range(NBUF):
            j = g + ph
            in_buf = in_bufs.at[ph]
            pos_buf = pos_bufs.at[ph]
            out_buf = out_bufs.at[ph]
            pltpu.make_async_copy(in_src(j), in_buf, in_sems.at[ph]).wait()
            pltpu.make_async_copy(pos_src(j), pos_buf, pos_sems.at[ph]).wait()

            @pl.when(g > 0)
            def _wait_out():
                pltpu.make_async_copy(out_buf, out_dst(j), out_sems.at[ph]).wait()

            @plsc.parallel_loop(0, 8 * WV, unroll=2)
            def _add(i):
                pp = i >> 4
                o = (i & (WV - 1)) * LANES
                pv = pos_buf[pp, pl.ds(o, LANES)]
                for bb in range(8):
                    out_buf[pp, bb, pl.ds(o, LANES)] = (
                        in_buf[bb, pp, pl.ds(o, LANES)] + pv)

            pltpu.async_copy(out_buf, out_dst(j), out_sems.at[ph])
            nj = jnp.minimum(j + NBUF, UPW - 1)
            pltpu.async_copy(in_src(nj), in_bufs.at[ph], in_sems.at[ph])
            pltpu.async_copy(pos_src(nj), pos_bufs.at[ph], pos_sems.at[ph])

    for ph in range(NBUF):
        pltpu.make_async_copy(in_src(UPW - 1), in_bufs.at[ph], in_sems.at[ph]).wait()
        pltpu.make_async_copy(pos_src(UPW - 1), pos_bufs.at[ph], pos_sems.at[ph]).wait()
        pltpu.make_async_copy(out_bufs.at[ph], out_dst(UPW - NBUF + ph), out_sems.at[ph]).wait()


def kernel(inputs, class_embed, pos_table):
    pos_sh = pos_table[1:]
    row0 = class_embed.reshape(D_MODEL) + pos_table[0]
    mesh = plsc.VectorSubcoreMesh(
        core_axis_name="c", subcore_axis_name="s", num_cores=2, num_subcores=16)
    out_phys = pl.kernel(
        _sc_body,
        out_type=jax.ShapeDtypeStruct((N_TOT, BATCH, D_MODEL), jnp.float32),
        mesh=mesh,
        scratch_types=[
            pltpu.VMEM((D_MODEL,), jnp.float32),
            pltpu.VMEM((8, D_MODEL), jnp.float32),
            pltpu.VMEM((NBUF, 8, 8, W), jnp.float32),
            pltpu.VMEM((NBUF, 8, W), jnp.float32),
            pltpu.VMEM((NBUF, 8, 8, W), jnp.float32),
            pltpu.SemaphoreType.DMA((NBUF,)),
            pltpu.SemaphoreType.DMA((NBUF,)),
            pltpu.SemaphoreType.DMA((NBUF,)),
        ],
    )(inputs, pos_sh, row0)
    return jnp.transpose(out_phys, (1, 0, 2))
